# baseline (speedup 1.0000x reference)
import jax
import jax.numpy as jnp
from jax.experimental import pallas as pl


def _final_node_kernel(a1_ref, nf_ref, df_ref, nb_ref, db_ref, h_ref,
                       gamma_ref, beta_ref, out_ref):
    eps = 1e-6
    hf = nf_ref[...] / (df_ref[...] + eps)
    hb = nb_ref[...] / (db_ref[...] + eps)
    x = a1_ref[...] + hf + hb
    mu = jnp.mean(x, axis=0, keepdims=True)
    var = jnp.mean((x - mu) ** 2, axis=0, keepdims=True)
    xn = gamma_ref[...] * (x - mu) / jnp.sqrt(var + 1e-5) + beta_ref[...]
    out_ref[...] = jnp.maximum(xn, 0.0) + h_ref[...]


def kernel(h, e, edge_index, WA1, bA1, WA2, bA2, WA3, bA3, WB1, bB1, WB2, bB2,
           WB3, bB3, gamma_h, beta_h, gamma_e, beta_e):
    n = h.shape[0]
    src = edge_index[0]
    dst = edge_index[1]

    A1h = h @ WA1.T + bA1
    A2h = h @ WA2.T + bA2
    A3h = h @ WA3.T + bA3
    B1h = h @ WB1.T + bB1
    B2h = h @ WB2.T + bB2
    C = e @ WB3.T + bB3

    x = B1h[src] + B2h[dst] + C
    mu = x.mean(axis=0)
    var = x.var(axis=0)
    xn = gamma_e * (x - mu) / jnp.sqrt(var + 1e-5) + beta_e
    sigma = jax.nn.sigmoid(jnp.maximum(xn, 0.0) + e)

    num_f = jax.ops.segment_sum(A2h[src] * sigma, dst, num_segments=n)
    den_f = jax.ops.segment_sum(sigma, dst, num_segments=n)
    num_b = jax.ops.segment_sum(A3h[dst] * sigma, src, num_segments=n)
    den_b = jax.ops.segment_sum(sigma, src, num_segments=n)

    h_out = pl.pallas_call(
        _final_node_kernel,
        out_shape=jax.ShapeDtypeStruct(h.shape, h.dtype),
    )(A1h, num_f, den_f, num_b, den_b, h,
      gamma_h.reshape(1, -1), beta_h.reshape(1, -1))
    return (h_out, e)


# speedup vs baseline: 1.2868x; 1.2868x over previous
import functools

import jax
import jax.numpy as jnp
from jax import lax
from jax.experimental import pallas as pl
from jax.experimental.pallas import tpu as pltpu
from jax.experimental.pallas import tpu_sc as plsc

N_NODES = 10000
N_EDGES = 320000
D = 128

CG = 32
NG = D // CG
CHUNK = 128
N_PAD = 10240
ROWS_PER_TILE = N_PAD // 16


def _seg_body(sigma_hbm, a2p_hbm, a3p_hbm, src2_hbm, dst2_hbm, zeros_hbm,
              nf_hbm, df_hbm, nb_hbm, db_hbm,
              fwd_p, fwd_d, bwd_p, bwd_d,
              idx_s, idx_d, idx_gs, idx_gd, a2b, a3b, sgb, flb, sem):
    c = lax.axis_index("c")
    s = lax.axis_index("s")
    n_rows = N_EDGES // CHUNK
    k_lo = (s * n_rows) // 16
    k_hi = ((s + 1) * n_rows) // 16
    row0 = s * ROWS_PER_TILE

    for g in range(2):
        gg = c * 2 + g
        col0 = gg * CG

        out_base = gg * N_PAD
        pltpu.sync_copy(zeros_hbm, flb)
        for acc in (fwd_p, fwd_d, bwd_p, bwd_d):
            pltpu.sync_copy(flb, acc.at[pl.ds(row0, ROWS_PER_TILE)])
        plsc.subcore_barrier()

        def chunk_body(k, _):
            pltpu.sync_copy(src2_hbm.at[k], idx_s)
            pltpu.sync_copy(dst2_hbm.at[k], idx_d)
            bias = gg * N_NODES
            for i in range(CHUNK // 16):
                sl = pl.ds(i * 16, 16)
                idx_gs[sl] = idx_s[sl] + bias
                idx_gd[sl] = idx_d[sl] + bias
            pltpu.async_copy(a2p_hbm.at[idx_gs], a2b, sem).wait()
            pltpu.async_copy(a3p_hbm.at[idx_gd], a3b, sem).wait()
            pltpu.sync_copy(
                sigma_hbm.at[pl.ds(gg * N_EDGES + k * CHUNK, CHUNK)], sgb)

            def mul_body(i, _):
                for h in range(CG // 16):
                    sl = pl.ds(h * 16, 16)
                    sg = sgb[i, sl]
                    a2b[i, sl] = a2b[i, sl] * sg
                    a3b[i, sl] = a3b[i, sl] * sg
                return _

            lax.fori_loop(0, CHUNK, mul_body, 0, unroll=4)

            pltpu.sync_copy(a2b, fwd_p.at[idx_d], add=True)
            pltpu.sync_copy(sgb, fwd_d.at[idx_d], add=True)
            pltpu.sync_copy(a3b, bwd_p.at[idx_s], add=True)
            pltpu.sync_copy(sgb, bwd_d.at[idx_s], add=True)
            return _

        lax.fori_loop(k_lo, k_hi, chunk_body, 0)
        plsc.subcore_barrier()

        for acc, out in ((fwd_p, nf_hbm), (fwd_d, df_hbm),
                         (bwd_p, nb_hbm), (bwd_d, db_hbm)):
            pltpu.sync_copy(acc.at[pl.ds(row0, ROWS_PER_TILE)], flb)
            pltpu.sync_copy(
                flb, out.at[pl.ds(out_base + row0, ROWS_PER_TILE)])
        plsc.subcore_barrier()


def _segment_sums(sigma, a2p, a3p, src2, dst2, zeros):
    out4 = jax.ShapeDtypeStruct((NG * N_PAD, CG), jnp.float32)
    mesh = plsc.VectorSubcoreMesh(core_axis_name="c", subcore_axis_name="s")
    f = pl.kernel(
        _seg_body,
        out_type=(out4, out4, out4, out4),
        mesh=mesh,
        scratch_types=[
            pltpu.VMEM_SHARED((N_PAD, CG), jnp.float32),
            pltpu.VMEM_SHARED((N_PAD, CG), jnp.float32),
            pltpu.VMEM_SHARED((N_PAD, CG), jnp.float32),
            pltpu.VMEM_SHARED((N_PAD, CG), jnp.float32),
            pltpu.VMEM((CHUNK,), jnp.int32),
            pltpu.VMEM((CHUNK,), jnp.int32),
            pltpu.VMEM((CHUNK,), jnp.int32),
            pltpu.VMEM((CHUNK,), jnp.int32),
            pltpu.VMEM((CHUNK, CG), jnp.float32),
            pltpu.VMEM((CHUNK, CG), jnp.float32),
            pltpu.VMEM((CHUNK, CG), jnp.float32),
            pltpu.VMEM((ROWS_PER_TILE, CG), jnp.float32),
            pltpu.SemaphoreType.DMA,
        ],
        compiler_params=pltpu.CompilerParams(use_tc_tiling_on_sc=False),
    )
    return f(sigma, a2p, a3p, src2, dst2, zeros)


def _final_node_kernel(a1_ref, nf_ref, df_ref, nb_ref, db_ref, h_ref,
                       gamma_ref, beta_ref, out_ref):
    eps = 1e-6
    hf = nf_ref[...] / (df_ref[...] + eps)
    hb = nb_ref[...] / (db_ref[...] + eps)
    x = a1_ref[...] + hf + hb
    mu = jnp.mean(x, axis=0, keepdims=True)
    var = jnp.mean((x - mu) ** 2, axis=0, keepdims=True)
    xn = gamma_ref[...] * (x - mu) / jnp.sqrt(var + 1e-5) + beta_ref[...]
    out_ref[...] = jnp.maximum(xn, 0.0) + h_ref[...]


def kernel(h, e, edge_index, WA1, bA1, WA2, bA2, WA3, bA3, WB1, bB1, WB2, bB2,
           WB3, bB3, gamma_h, beta_h, gamma_e, beta_e):
    src = edge_index[0].astype(jnp.int32)
    dst = edge_index[1].astype(jnp.int32)

    A1h = h @ WA1.T + bA1
    A2h = h @ WA2.T + bA2
    A3h = h @ WA3.T + bA3
    B1h = h @ WB1.T + bB1
    B2h = h @ WB2.T + bB2
    C = e @ WB3.T + bB3

    x = B1h[src] + B2h[dst] + C
    mu = x.mean(axis=0)
    var = x.var(axis=0)
    xn = gamma_e * (x - mu) / jnp.sqrt(var + 1e-5) + beta_e
    sigma = jax.nn.sigmoid(jnp.maximum(xn, 0.0) + e)
    sigp = jnp.concatenate([sigma[:, i * CG:(i + 1) * CG] for i in range(NG)],
                           0)

    a2p = jnp.concatenate([A2h[:, i * CG:(i + 1) * CG] for i in range(NG)], 0)
    a3p = jnp.concatenate([A3h[:, i * CG:(i + 1) * CG] for i in range(NG)], 0)
    src2 = src.reshape(N_EDGES // CHUNK, CHUNK)
    dst2 = dst.reshape(N_EDGES // CHUNK, CHUNK)
    zeros = jnp.zeros((ROWS_PER_TILE, CG), jnp.float32)

    nf_p, df_p, nb_p, db_p = _segment_sums(sigp, a2p, a3p, src2, dst2, zeros)

    def unpack(t):
        return jnp.concatenate(
            [t[i * N_PAD:i * N_PAD + N_NODES] for i in range(NG)], axis=1)

    num_f, den_f, num_b, den_b = (unpack(nf_p), unpack(df_p), unpack(nb_p),
                                  unpack(db_p))

    h_out = pl.pallas_call(
        _final_node_kernel,
        out_shape=jax.ShapeDtypeStruct(h.shape, h.dtype),
    )(A1h, num_f, den_f, num_b, den_b, h,
      gamma_h.reshape(1, -1), beta_h.reshape(1, -1))
    return (h_out, e)
